# baseline (speedup 1.0000x reference)
import functools

import jax
import jax.numpy as jnp
from jax import lax
from jax.experimental import pallas as pl
from jax.experimental.pallas import tpu as pltpu
from jax.experimental.pallas import tpu_sc as plsc

N_NODES = 10000
N_PAD = 10112
NPK = 2560
D = 128
HD = 64
NC = 2
NS = 16
LANES = 16
B = 128
CH = 2
ROWS_PER_TILE = N_PAD // NS
XROWS_PER_TILE = NPK // NS


def _sc_aggregate(xp, src3, dst3, w13, w23, nb):
    nchunks = nb // CH
    mesh = plsc.VectorSubcoreMesh(
        core_axis_name="c", subcore_axis_name="s", num_cores=NC, num_subcores=NS
    )

    @functools.partial(
        pl.kernel,
        out_type=jax.ShapeDtypeStruct((NC, N_PAD, D), jnp.float32),
        mesh=mesh,
        scratch_types=[
            pltpu.VMEM_SHARED((N_PAD, D), jnp.float32),
            pltpu.VMEM_SHARED((NPK, D), jnp.float32),
            [pltpu.VMEM((CH, B), jnp.int32)] * 2,
            [pltpu.VMEM((CH, B), jnp.int32)] * 2,
            [pltpu.VMEM((CH, B), jnp.int32)] * 2,
            [pltpu.VMEM((CH, B), jnp.float32)] * 2,
            [pltpu.VMEM((CH, B), jnp.float32)] * 2,
            pltpu.VMEM((B, D), jnp.float32),
            [pltpu.SemaphoreType.DMA] * 2,
            pltpu.SemaphoreType.DMA,
            pltpu.SemaphoreType.DMA,
        ],
    )
    def k(xp_hbm, src_hbm, dst_hbm, w1_hbm, w2_hbm, out_hbm,
          acc_sh, xp_sh, srcc, src2c, dstc, w1c, w2c, vals,
          semi, semg, sems):
        c = lax.axis_index("c")
        s = lax.axis_index("s")
        row0 = s * ROWS_PER_TILE
        xrow0 = s * XROWS_PER_TILE

        pltpu.sync_copy(xp_hbm.at[c, pl.ds(xrow0, XROWS_PER_TILE)],
                        xp_sh.at[pl.ds(xrow0, XROWS_PER_TILE)])

        def zrow(j, carry):
            for kk in range(D // LANES):
                vals[j, pl.ds(kk * LANES, LANES)] = jnp.zeros((LANES,), jnp.float32)
            return carry
        lax.fori_loop(0, B, zrow, 0)
        nfull = ROWS_PER_TILE // B
        for i in range(nfull):
            pltpu.sync_copy(vals, acc_sh.at[pl.ds(row0 + i * B, B)])
        rem = ROWS_PER_TILE - nfull * B
        pltpu.sync_copy(vals.at[pl.ds(0, rem)],
                        acc_sh.at[pl.ds(row0 + nfull * B, rem)])
        plsc.subcore_barrier()

        def idx_start(ci, q):
            pltpu.async_copy(src_hbm.at[s, pl.ds(ci * CH, CH)], srcc[q], semi[q])
            pltpu.async_copy(dst_hbm.at[s, pl.ds(ci * CH, CH)], dstc[q], semi[q])
            pltpu.async_copy(w1_hbm.at[s, pl.ds(ci * CH, CH)], w1c[q], semi[q])
            pltpu.async_copy(w2_hbm.at[s, pl.ds(ci * CH, CH)], w2c[q], semi[q])

        def idx_wait(ci, q):
            pltpu.make_async_copy(src_hbm.at[s, pl.ds(ci * CH, CH)], srcc[q], semi[q]).wait()
            pltpu.make_async_copy(dst_hbm.at[s, pl.ds(ci * CH, CH)], dstc[q], semi[q]).wait()
            pltpu.make_async_copy(w1_hbm.at[s, pl.ds(ci * CH, CH)], w1c[q], semi[q]).wait()
            pltpu.make_async_copy(w2_hbm.at[s, pl.ds(ci * CH, CH)], w2c[q], semi[q]).wait()

        def src2_fill(q):
            def f(i, carry):
                for kk in range(B // LANES):
                    sl = pl.ds(kk * LANES, LANES)
                    src2c[q][i, sl] = lax.shift_right_logical(srcc[q][i, sl], 2)
                return carry
            lax.fori_loop(0, CH, f, 0)

        def build(q, j):
            @plsc.parallel_loop(0, B // LANES)
            def grp(g):
                sl16 = pl.ds(g * LANES, LANES)
                sgroup = srcc[q][j, sl16]
                pgroup = lax.bitwise_and(sgroup, 3) * 32
                w1g = w1c[q][j, sl16]
                w2g = w2c[q][j, sl16]
                for jj in range(LANES):
                    off = pl.multiple_of(pgroup[jj], 32)
                    w1j = w1g[jj]
                    w2j = w2g[jj]
                    r = g * LANES + jj
                    wa = lax.bitcast_convert_type(
                        vals[r, pl.ds(off, LANES)], jnp.int32)
                    wb = lax.bitcast_convert_type(
                        vals[r, pl.ds(off + LANES, LANES)], jnp.int32)
                    los = []
                    his = []
                    for wvec in (wa, wb):
                        los.append(lax.bitcast_convert_type(
                            lax.shift_left(wvec, 16), jnp.float32))
                        his.append(lax.bitcast_convert_type(
                            lax.bitwise_and(wvec, jnp.int32(-65536)),
                            jnp.float32))
                    for kk in range(2):
                        c0 = pl.ds(kk * 32, LANES)
                        c1 = pl.ds(kk * 32 + LANES, LANES)
                        vals[r, c0] = los[kk] * w1j
                        vals[r, c1] = his[kk] * w1j
                    for kk in range(2):
                        c2 = pl.ds(HD + kk * 32, LANES)
                        c3 = pl.ds(HD + kk * 32 + LANES, LANES)
                        vals[r, c2] = los[kk] * w2j
                        vals[r, c3] = his[kk] * w2j

        idx_start(0, 0)
        idx_start(1, 1)

        def body(t, carry):
            for ch in range(2):
                ci = t * 2 + ch
                for j in range(CH):
                    if j == 0:
                        idx_wait(ci, ch)
                        src2_fill(ch)
                    pltpu.async_copy(
                        xp_sh.at[src2c[ch].at[j]], vals, semg).wait()
                    build(ch, j)
                    cp = pltpu.async_copy(
                        vals, acc_sh.at[dstc[ch].at[j]], sems, add=True)
                    cp.wait()
                    if j == CH - 1:
                        @pl.when(ci + 2 < nchunks)
                        def _():
                            idx_start(ci + 2, ch)
            return carry
        lax.fori_loop(0, nchunks // 2, body, 0)

        plsc.subcore_barrier()
        pltpu.sync_copy(acc_sh.at[pl.ds(row0, ROWS_PER_TILE)],
                        out_hbm.at[c, pl.ds(row0, ROWS_PER_TILE)])

    return k(xp, src3, dst3, w13, w23)


def _tc_combine(agg, W0p, W1p, bias):
    BM = 1000
    grid = (N_NODES // BM,)

    def body(a0, a1, w0, w1, bref, o):
        o[:, :] = (
            jnp.dot(a0[0], w0[:, :], preferred_element_type=jnp.float32)
            + jnp.dot(a1[0], w1[:, :], preferred_element_type=jnp.float32)
            + bref[:, :]
        )

    return pl.pallas_call(
        body,
        grid=grid,
        in_specs=[
            pl.BlockSpec((1, BM, D), lambda i: (0, i, 0)),
            pl.BlockSpec((1, BM, D), lambda i: (1, i, 0)),
            pl.BlockSpec((D, D), lambda i: (0, 0)),
            pl.BlockSpec((D, D), lambda i: (0, 0)),
            pl.BlockSpec((1, D), lambda i: (0, 0)),
        ],
        out_specs=pl.BlockSpec((BM, D), lambda i: (i, 0)),
        out_shape=jax.ShapeDtypeStruct((N_NODES, D), jnp.float32),
    )(agg, agg, W0p, W1p, bias)


def kernel(x, edge_index, edge_weight_src_to_tgt, edge_weight_tgt_to_src,
           W_src_to_dst, W_dst_to_src, b_src_to_dst, b_dst_to_src):
    E = edge_index.shape[1]
    gran = NS * B * CH * 2
    epad = -(-E // gran) * gran
    nb = epad // (NS * B)
    pad = epad - E
    pad_src = (jnp.arange(pad, dtype=jnp.int32) * 16) % N_NODES
    src = jnp.concatenate([edge_index[0], pad_src]).reshape(NS, nb, B)
    dst = jnp.pad(edge_index[1], (0, pad)).reshape(NS, nb, B)
    w1 = jnp.pad(edge_weight_src_to_tgt[:, 0], (0, pad)).reshape(NS, nb, B)
    w2 = jnp.pad(edge_weight_tgt_to_src[:, 0], (0, pad)).reshape(NS, nb, B)
    xpad = jnp.pad(x, ((0, 4 * NPK - N_NODES), (0, 0)))
    xp_bf = jnp.stack([
        xpad[:, :HD].reshape(NPK, 4 * HD),
        xpad[:, HD:].reshape(NPK, 4 * HD),
    ]).astype(jnp.bfloat16)
    xp = jax.lax.bitcast_convert_type(jax.lax.bitcast_convert_type(
        xp_bf.reshape(NC, NPK, D, 2), jnp.int32), jnp.float32)
    agg = _sc_aggregate(xp, src, dst, w1, w2, nb)
    perm = jnp.concatenate([
        jnp.concatenate([kk * 32 + 2 * jnp.arange(16, dtype=jnp.int32),
                         kk * 32 + 2 * jnp.arange(16, dtype=jnp.int32) + 1])
        for kk in range(2)
    ])
    W0p = jnp.concatenate([W_src_to_dst[perm], W_dst_to_src[perm]], axis=0)
    W1p = jnp.concatenate([W_src_to_dst[HD + perm], W_dst_to_src[HD + perm]], axis=0)
    bias = (b_src_to_dst + b_dst_to_src).reshape(1, D)
    return _tc_combine(agg, W0p, W1p, bias)


# speedup vs baseline: 1.3751x; 1.3751x over previous
import functools

import jax
import jax.numpy as jnp
from jax import lax
from jax.experimental import pallas as pl
from jax.experimental.pallas import tpu as pltpu
from jax.experimental.pallas import tpu_sc as plsc

N_NODES = 10000
N_PAD = 10112
D = 128
NC = 2
NS = 16
LANES = 16
B = 128
ROWS_PER_TILE = N_PAD // NS


def _sc_aggregate(x, src, dst, w1, w2, nbatches):
    mesh = plsc.VectorSubcoreMesh(
        core_axis_name="c", subcore_axis_name="s", num_cores=NC, num_subcores=NS
    )

    @functools.partial(
        pl.kernel,
        out_type=jax.ShapeDtypeStruct((NC, N_PAD, D), jnp.float32),
        mesh=mesh,
        scratch_types=[
            pltpu.VMEM_SHARED((N_PAD, D), jnp.float32),
            pltpu.VMEM((B,), jnp.int32),
            pltpu.VMEM((B,), jnp.int32),
            pltpu.VMEM((B,), jnp.float32),
            pltpu.VMEM((B, D), jnp.float32),
            pltpu.SemaphoreType.DMA,
        ],
    )
    def k(x_hbm, src_hbm, dst_hbm, w1_hbm, w2_hbm, out_hbm,
          acc_sh, srcv, dstv, wv, rows, sem):
        c = lax.axis_index("c")
        s = lax.axis_index("s")
        row0 = s * ROWS_PER_TILE

        def zrow(j, carry):
            for kk in range(D // LANES):
                rows[j, pl.ds(kk * LANES, LANES)] = jnp.zeros((LANES,), jnp.float32)
            return carry
        lax.fori_loop(0, B, zrow, 0)
        for i in range(4):
            pltpu.sync_copy(rows, acc_sh.at[pl.ds(row0 + i * B, B)])
        pltpu.sync_copy(rows.at[pl.ds(0, ROWS_PER_TILE - 4 * B)],
                        acc_sh.at[pl.ds(row0 + 4 * B, ROWS_PER_TILE - 4 * B)])
        plsc.subcore_barrier()

        base = s * (nbatches * B)

        def body(b, carry):
            off = base + b * B
            pltpu.sync_copy(src_hbm.at[pl.ds(off, B)], srcv)
            pltpu.sync_copy(dst_hbm.at[pl.ds(off, B)], dstv)

            @pl.when(c == 0)
            def _():
                pltpu.sync_copy(w1_hbm.at[pl.ds(off, B)], wv)

            @pl.when(c != 0)
            def _():
                pltpu.sync_copy(w2_hbm.at[pl.ds(off, B)], wv)

            pltpu.async_copy(x_hbm.at[srcv], rows, sem).wait()

            @plsc.parallel_loop(0, B // LANES)
            def scale(g):
                wgroup = wv[pl.ds(g * LANES, LANES)]
                for jj in range(LANES):
                    wj = wgroup[jj]
                    j = g * LANES + jj
                    for kk in range(D // LANES):
                        sl = pl.ds(kk * LANES, LANES)
                        rows[j, sl] = rows[j, sl] * wj

            pltpu.sync_copy(rows, acc_sh.at[dstv], add=True)
            return carry
        lax.fori_loop(0, nbatches, body, 0)

        plsc.subcore_barrier()
        pltpu.sync_copy(acc_sh.at[pl.ds(row0, ROWS_PER_TILE)],
                        out_hbm.at[c, pl.ds(row0, ROWS_PER_TILE)])

    return k(x, src, dst, w1, w2)


def _tc_combine(agg, W1, W2, bias):
    BM = 1000
    grid = (N_NODES // BM,)

    def body(a0, a1, w1, w2, bref, o):
        o[:, :] = (
            jnp.dot(a0[0], w1[:, :], preferred_element_type=jnp.float32)
            + jnp.dot(a1[0], w2[:, :], preferred_element_type=jnp.float32)
            + bref[:, :]
        )

    return pl.pallas_call(
        body,
        grid=grid,
        in_specs=[
            pl.BlockSpec((1, BM, D), lambda i: (0, i, 0)),
            pl.BlockSpec((1, BM, D), lambda i: (1, i, 0)),
            pl.BlockSpec((D, D), lambda i: (0, 0)),
            pl.BlockSpec((D, D), lambda i: (0, 0)),
            pl.BlockSpec((1, D), lambda i: (0, 0)),
        ],
        out_specs=pl.BlockSpec((BM, D), lambda i: (i, 0)),
        out_shape=jax.ShapeDtypeStruct((N_NODES, D), jnp.float32),
    )(agg, agg, W1, W2, bias)


def kernel(x, edge_index, edge_weight_src_to_tgt, edge_weight_tgt_to_src,
           W_src_to_dst, W_dst_to_src, b_src_to_dst, b_dst_to_src):
    E = edge_index.shape[1]
    nbatches = -(-E // (NS * B))
    epad = NS * B * nbatches
    pad = epad - E
    pad_src = (jnp.arange(pad, dtype=jnp.int32) * 16) % N_NODES
    src = jnp.concatenate([edge_index[0], pad_src])
    dst = jnp.pad(edge_index[1], (0, pad))
    w1 = jnp.pad(edge_weight_src_to_tgt[:, 0], (0, pad))
    w2 = jnp.pad(edge_weight_tgt_to_src[:, 0], (0, pad))
    agg = _sc_aggregate(x, src, dst, w1, w2, nbatches)
    bias = (b_src_to_dst + b_dst_to_src).reshape(1, D)
    return _tc_combine(agg, W_src_to_dst, W_dst_to_src, bias)


# speedup vs baseline: 1.9495x; 1.4177x over previous
import functools

import jax
import jax.numpy as jnp
from jax import lax
from jax.experimental import pallas as pl
from jax.experimental.pallas import tpu as pltpu
from jax.experimental.pallas import tpu_sc as plsc

N_NODES = 10000
N_PAD = 10112
D = 128
NC = 2
NS = 16
LANES = 16
B = 128
ROWS_PER_TILE = N_PAD // NS


def _sc_aggregate(x, src, dst, w1, w2, nbatches):
    mesh = plsc.VectorSubcoreMesh(
        core_axis_name="c", subcore_axis_name="s", num_cores=NC, num_subcores=NS
    )

    @functools.partial(
        pl.kernel,
        out_type=jax.ShapeDtypeStruct((NC, N_PAD, D), jnp.float32),
        mesh=mesh,
        scratch_types=[
            pltpu.VMEM_SHARED((N_PAD, D), jnp.float32),
            pltpu.VMEM((2, B), jnp.int32),
            pltpu.VMEM((2, B), jnp.int32),
            pltpu.VMEM((2, B), jnp.float32),
            pltpu.VMEM((B, D), jnp.float32),
            pltpu.SemaphoreType.DMA,
            [pltpu.SemaphoreType.DMA] * 2,
        ],
    )
    def k(x_hbm, src_hbm, dst_hbm, w1_hbm, w2_hbm, out_hbm,
          acc_sh, srcv, dstv, wv, rows, sem, semi):
        c = lax.axis_index("c")
        s = lax.axis_index("s")
        row0 = s * ROWS_PER_TILE

        def zrow(j, carry):
            for kk in range(D // LANES):
                rows[j, pl.ds(kk * LANES, LANES)] = jnp.zeros((LANES,), jnp.float32)
            return carry
        lax.fori_loop(0, B, zrow, 0)
        for i in range(4):
            pltpu.sync_copy(rows, acc_sh.at[pl.ds(row0 + i * B, B)])
        pltpu.sync_copy(rows.at[pl.ds(0, ROWS_PER_TILE - 4 * B)],
                        acc_sh.at[pl.ds(row0 + 4 * B, ROWS_PER_TILE - 4 * B)])
        plsc.subcore_barrier()

        base = s * (nbatches * B)

        def idx_start(b, p):
            off = base + b * B
            pltpu.async_copy(src_hbm.at[pl.ds(off, B)], srcv.at[p], semi[p])
            pltpu.async_copy(dst_hbm.at[pl.ds(off, B)], dstv.at[p], semi[p])

            @pl.when(c == 0)
            def _():
                pltpu.async_copy(w1_hbm.at[pl.ds(off, B)], wv.at[p], semi[p])

            @pl.when(c != 0)
            def _():
                pltpu.async_copy(w2_hbm.at[pl.ds(off, B)], wv.at[p], semi[p])

        def idx_wait(b, p):
            off = base + b * B
            pltpu.make_async_copy(
                src_hbm.at[pl.ds(off, B)], srcv.at[p], semi[p]).wait()
            pltpu.make_async_copy(
                dst_hbm.at[pl.ds(off, B)], dstv.at[p], semi[p]).wait()
            pltpu.make_async_copy(
                w1_hbm.at[pl.ds(off, B)], wv.at[p], semi[p]).wait()

        idx_start(0, 0)

        def body(t, carry):
            for p in range(2):
                b = t * 2 + p
                idx_wait(b, p)

                @pl.when(b + 1 < nbatches)
                def _():
                    idx_start(b + 1, 1 - p)

                pltpu.async_copy(x_hbm.at[srcv.at[p]], rows, sem).wait()

                @plsc.parallel_loop(0, B // LANES)
                def scale(g):
                    wgroup = wv[p, pl.ds(g * LANES, LANES)]
                    for jj in range(LANES):
                        wj = wgroup[jj]
                        j = g * LANES + jj
                        for kk in range(D // LANES):
                            sl = pl.ds(kk * LANES, LANES)
                            rows[j, sl] = rows[j, sl] * wj

                pltpu.sync_copy(rows, acc_sh.at[dstv.at[p]], add=True)
            return carry
        lax.fori_loop(0, nbatches // 2, body, 0)

        plsc.subcore_barrier()
        pltpu.sync_copy(acc_sh.at[pl.ds(row0, ROWS_PER_TILE)],
                        out_hbm.at[c, pl.ds(row0, ROWS_PER_TILE)])

    return k(x, src, dst, w1, w2)


def _tc_combine(agg, W1, W2, bias):
    BM = 1000
    grid = (N_NODES // BM,)

    def body(a0, a1, w1, w2, bref, o):
        o[:, :] = (
            jnp.dot(a0[0], w1[:, :], preferred_element_type=jnp.float32)
            + jnp.dot(a1[0], w2[:, :], preferred_element_type=jnp.float32)
            + bref[:, :]
        )

    return pl.pallas_call(
        body,
        grid=grid,
        in_specs=[
            pl.BlockSpec((1, BM, D), lambda i: (0, i, 0)),
            pl.BlockSpec((1, BM, D), lambda i: (1, i, 0)),
            pl.BlockSpec((D, D), lambda i: (0, 0)),
            pl.BlockSpec((D, D), lambda i: (0, 0)),
            pl.BlockSpec((1, D), lambda i: (0, 0)),
        ],
        out_specs=pl.BlockSpec((BM, D), lambda i: (i, 0)),
        out_shape=jax.ShapeDtypeStruct((N_NODES, D), jnp.float32),
    )(agg, agg, W1, W2, bias)


def kernel(x, edge_index, edge_weight_src_to_tgt, edge_weight_tgt_to_src,
           W_src_to_dst, W_dst_to_src, b_src_to_dst, b_dst_to_src):
    E = edge_index.shape[1]
    nbatches = 2 * (-(-E // (NS * B * 2)))
    epad = NS * B * nbatches
    pad = epad - E
    pad_src = (jnp.arange(pad, dtype=jnp.int32) * 16) % N_NODES
    src = jnp.concatenate([edge_index[0], pad_src])
    dst = jnp.pad(edge_index[1], (0, pad))
    w1 = jnp.pad(edge_weight_src_to_tgt[:, 0], (0, pad))
    w2 = jnp.pad(edge_weight_tgt_to_src[:, 0], (0, pad))
    agg = _sc_aggregate(x, src, dst, w1, w2, nbatches)
    bias = (b_src_to_dst + b_dst_to_src).reshape(1, D)
    return _tc_combine(agg, W_src_to_dst, W_dst_to_src, bias)
